# baseline (speedup 1.0000x reference)
import functools

import jax
import jax.numpy as jnp
from jax import lax
from jax.experimental import pallas as pl
from jax.experimental.pallas import tpu as pltpu
from jax.experimental.pallas import tpu_sc as plsc

N = 10000
C = 128
D = 16
H = 4 * C
E = 320000
G = 16
EPS = 1e-5

_LP = (3.50755205e-06, 9.99792436e-01, -4.96977911e-01, 3.14590535e-01,
       -1.88782674e-01, 8.17268084e-02, -1.72080611e-02)

_NC = 2
_NS = 16
_NW = _NC * _NS
_CH = 40
_NCHUNK = E // _CH
_CPW = _NCHUNK // _NW
_ZONE = 624


_NB = 10
_BN = N // _NB
_EB = 160
_BE = E // _EB


def _node_proj_body(x_ref, wd_ref, ws_ref, tg_ref, ts_ref):
    xb = x_ref[...]
    tg_ref[...] = jnp.dot(xb, wd_ref[...], preferred_element_type=jnp.float32)
    ts_ref[...] = jnp.dot(xb, ws_ref[...], preferred_element_type=jnp.float32)


def _node_proj(x, wd, ws):
    return pl.pallas_call(
        _node_proj_body,
        grid=(_NB,),
        in_specs=[
            pl.BlockSpec((_BN, C), lambda i: (i, 0)),
            pl.BlockSpec((C, 2 * C), lambda i: (0, 0)),
            pl.BlockSpec((C, 2 * C), lambda i: (0, 0)),
        ],
        out_specs=[
            pl.BlockSpec((_BN, 2 * C), lambda i: (i, 0)),
            pl.BlockSpec((_BN, 2 * C), lambda i: (i, 0)),
        ],
        out_shape=[
            jax.ShapeDtypeStruct((N, 2 * C), jnp.float32),
            jax.ShapeDtypeStruct((N, 2 * C), jnp.float32),
        ],
    )(x, wd, ws)


def _edge_proj_body(ea_ref, we_ref, be_ref, t0_ref, t1_ref, t2_ref):
    eb = ea_ref[...]
    t = jnp.dot(eb, we_ref[...], preferred_element_type=jnp.float32) + be_ref[...]
    t0_ref[...] = t[:, 0:256]
    t1_ref[...] = t[:, 256:512]
    t2_ref[...] = t[:, 512:768]


def _edge_proj(edge_attr, we_all, be_all):
    return pl.pallas_call(
        _edge_proj_body,
        grid=(_EB,),
        in_specs=[
            pl.BlockSpec((_BE, D), lambda i: (i, 0)),
            pl.BlockSpec((D, 768), lambda i: (0, 0)),
            pl.BlockSpec((1, 768), lambda i: (0, 0)),
        ],
        out_specs=[pl.BlockSpec((_BE, 256), lambda i: (i, 0))] * 3,
        out_shape=[jax.ShapeDtypeStruct((E, 256), jnp.float32)] * 3,
    )(edge_attr, we_all, be_all)


def _stats_body(x_ref, p0_ref, p1_ref, x1_ref, m_ref, s_ref):
    i = pl.program_id(0)
    x1 = x_ref[...] + p0_ref[...] + p1_ref[...]
    x1_ref[...] = x1
    mb = jnp.sum(x1, axis=0, keepdims=True)
    sb = lax.dot_general(x1, x1, (((0,), (0,)), ((), ())),
                         preferred_element_type=jnp.float32)

    @pl.when(i == 0)
    def _():
        m_ref[...] = mb
        s_ref[...] = sb

    @pl.when(i > 0)
    def _():
        m_ref[...] += mb
        s_ref[...] += sb


def _stats(x, p0, p1):
    return pl.pallas_call(
        _stats_body,
        grid=(_NB,),
        in_specs=[pl.BlockSpec((_BN, C), lambda i: (i, 0))] * 3,
        out_specs=[
            pl.BlockSpec((_BN, C), lambda i: (i, 0)),
            pl.BlockSpec((1, C), lambda i: (0, 0)),
            pl.BlockSpec((C, C), lambda i: (0, 0)),
        ],
        out_shape=[
            jax.ShapeDtypeStruct((N, C), jnp.float32),
            jax.ShapeDtypeStruct((1, C), jnp.float32),
            jax.ShapeDtypeStruct((C, C), jnp.float32),
        ],
    )(x, p0, p1)


def _mlp_body(x1_ref, m_ref, s_ref, w1_ref, b1_ref, bnw_ref, bnb_ref,
              w2_ref, b2_ref, nb_ref, x2_ref, gst_ref, scale_v, shift_v):
    i = pl.program_id(0)

    @pl.when(i == 0)
    def _():
        w1 = w1_ref[...]
        mw = jnp.dot(m_ref[...] / N, w1, preferred_element_type=jnp.float32)
        sw = jnp.dot(s_ref[...], w1, preferred_element_type=jnp.float32)
        q = jnp.sum(w1 * sw, axis=0, keepdims=True)
        b1 = b1_ref[...]
        mu = mw + b1
        ex2 = q / N + 2.0 * b1 * mw + b1 * b1
        var = ex2 - mu * mu
        scale = bnw_ref[...] * lax.rsqrt(var + EPS)
        scale_v[...] = scale
        shift_v[...] = bnb_ref[...] - mw * scale

    x1 = x1_ref[...]
    heff = jnp.dot(x1, w1_ref[...], preferred_element_type=jnp.float32)
    hn = jnp.maximum(heff * scale_v[...] + shift_v[...], 0.0)
    x2 = x1 + jnp.dot(hn, w2_ref[...], preferred_element_type=jnp.float32) + b2_ref[...]
    x2_ref[...] = x2

    nb = nb_ref[0, 0, :]
    oh = (nb[:, None] == lax.broadcasted_iota(jnp.int32, (1, C), 1)
          ).astype(jnp.float32)
    s1 = jnp.dot(jnp.sum(x2, axis=1)[None, :], oh,
                 preferred_element_type=jnp.float32)
    s2 = jnp.dot(jnp.sum(x2 * x2, axis=1)[None, :], oh,
                 preferred_element_type=jnp.float32)
    dg = jnp.sum(oh, axis=0, keepdims=True)
    row = jnp.concatenate([s1, s2, dg], axis=0)

    @pl.when(i == 0)
    def _():
        gst_ref[...] = row

    @pl.when(i > 0)
    def _():
        gst_ref[...] += row


def _mlp(x1, m, s, w1, b1, bnw, bnb, w2, b2, nb3):
    return pl.pallas_call(
        _mlp_body,
        grid=(_NB,),
        in_specs=[
            pl.BlockSpec((_BN, C), lambda i: (i, 0)),
            pl.BlockSpec((1, C), lambda i: (0, 0)),
            pl.BlockSpec((C, C), lambda i: (0, 0)),
            pl.BlockSpec((C, H), lambda i: (0, 0)),
            pl.BlockSpec((1, H), lambda i: (0, 0)),
            pl.BlockSpec((1, H), lambda i: (0, 0)),
            pl.BlockSpec((1, H), lambda i: (0, 0)),
            pl.BlockSpec((H, C), lambda i: (0, 0)),
            pl.BlockSpec((1, C), lambda i: (0, 0)),
            pl.BlockSpec((1, 1, _BN), lambda i: (i, 0, 0)),
        ],
        out_specs=[
            pl.BlockSpec((_BN, C), lambda i: (i, 0)),
            pl.BlockSpec((3, C), lambda i: (0, 0)),
        ],
        out_shape=[
            jax.ShapeDtypeStruct((N, C), jnp.float32),
            jax.ShapeDtypeStruct((3, C), jnp.float32),
        ],
        scratch_shapes=[
            pltpu.VMEM((1, H), jnp.float32),
            pltpu.VMEM((1, H), jnp.float32),
        ],
    )(x1, m, s, w1, b1, bnw, bnb, w2, b2, nb3)


def _ln_body(x2_ref, gst_ref, nb_ref, lnw_ref, lnb_ref, out_ref):
    s1 = gst_ref[0:1, :]
    s2 = gst_ref[1:2, :]
    dg = gst_ref[2:3, :]
    norm = jnp.maximum(dg, 1.0) * C
    mean = s1 / norm
    var = (s2 - 2.0 * mean * s1 + mean * mean * dg * C) / norm
    inv = lax.rsqrt(var + EPS)

    nb = nb_ref[0, 0, :]
    oh = (nb[:, None] == lax.broadcasted_iota(jnp.int32, (1, C), 1)
          ).astype(jnp.float32)
    mean_n = jnp.sum(oh * mean, axis=1, keepdims=True)
    inv_n = jnp.sum(oh * inv, axis=1, keepdims=True)
    x2 = x2_ref[...]
    out_ref[...] = (x2 - mean_n) * inv_n * lnw_ref[...] + lnb_ref[...]


def _ln_apply(x2, gst, nb3, lnw, lnb):
    return pl.pallas_call(
        _ln_body,
        grid=(_NB,),
        in_specs=[
            pl.BlockSpec((_BN, C), lambda i: (i, 0)),
            pl.BlockSpec((3, C), lambda i: (0, 0)),
            pl.BlockSpec((1, 1, _BN), lambda i: (i, 0, 0)),
            pl.BlockSpec((1, C), lambda i: (0, 0)),
            pl.BlockSpec((1, C), lambda i: (0, 0)),
        ],
        out_specs=pl.BlockSpec((_BN, C), lambda i: (i, 0)),
        out_shape=jax.ShapeDtypeStruct((N, C), jnp.float32),
    )(x2, gst, nb3, lnw, lnb)


def _sc_edge_body(tg_hbm, ts_hbm, te_hbm, dst_hbm, src_hbm,
                  p0_hbm, p1_hbm,
                  dsti_v, srci_v, grow_v, srow_v, te_v, msg_v,
                  aggr_sh, sem):
    cid = lax.axis_index("c")
    sid = lax.axis_index("s")
    wid = cid * _NS + sid

    def _zrow(r, _):
        for jj in range(C // 16):
            msg_v[r, pl.ds(jj * 16, 16)] = jnp.zeros((16,), jnp.float32)
        return _

    lax.fori_loop(0, _CH, _zrow, None)
    zbase = sid * _ZONE
    for k in range(_ZONE // _CH):
        pltpu.sync_copy(msg_v, aggr_sh.at[pl.ds(zbase + k * _CH, _CH)])
    _rem = _ZONE % _CH
    if _rem:
        pltpu.sync_copy(msg_v.at[pl.ds(0, _rem)],
                        aggr_sh.at[pl.ds(zbase + _ZONE - _rem, _rem)])

    @pl.when(sid == _NS - 1)
    def _():
        pltpu.sync_copy(msg_v.at[pl.ds(0, 16)],
                        aggr_sh.at[pl.ds(_NS * _ZONE, 16)])

    plsc.subcore_barrier()

    nch = _CPW
    start = wid * _CPW

    def _chunk(j, _):
        ch = start + j
        pltpu.sync_copy(dst_hbm.at[pl.ds(ch * _CH, _CH)], dsti_v)
        pltpu.sync_copy(src_hbm.at[pl.ds(ch * _CH, _CH)], srci_v)
        cp1 = pltpu.async_copy(tg_hbm.at[dsti_v], grow_v, sem)
        cp2 = pltpu.async_copy(ts_hbm.at[srci_v], srow_v, sem)
        cp3 = pltpu.async_copy(te_hbm.at[pl.ds(ch * _CH, _CH)], te_v, sem)
        cp1.wait()
        cp2.wait()
        cp3.wait()

        def _edge(e, _2):
            for jj in range(C // 16):
                o = jj * 16
                g = (grow_v[e, pl.ds(o, 16)] + srow_v[e, pl.ds(o, 16)]
                     + te_v[e, pl.ds(o, 16)])
                s = (grow_v[e, pl.ds(C + o, 16)] + srow_v[e, pl.ds(C + o, 16)]
                     + te_v[e, pl.ds(C + o, 16)])
                sig = 1.0 / (1.0 + jnp.exp(-g))
                t = jnp.exp(-jnp.abs(s))
                p = jnp.full((16,), _LP[6], jnp.float32)
                for cf in (_LP[5], _LP[4], _LP[3], _LP[2], _LP[1], _LP[0]):
                    p = p * t + cf
                sp = jnp.maximum(s, 0.0) + p
                msg_v[e, pl.ds(o, 16)] = sig * sp
            return _2

        lax.fori_loop(0, _CH, _edge, None)
        pltpu.sync_copy(msg_v, aggr_sh.at[dsti_v], add=True)
        return _

    lax.fori_loop(0, nch, _chunk, None)
    plsc.subcore_barrier()

    @pl.when(cid == 0)
    def _():
        pltpu.sync_copy(aggr_sh.at[pl.ds(zbase, _ZONE)],
                        p0_hbm.at[pl.ds(zbase, _ZONE)])

        @pl.when(sid == _NS - 1)
        def _():
            pltpu.sync_copy(aggr_sh.at[pl.ds(_NS * _ZONE, 16)],
                            p0_hbm.at[pl.ds(_NS * _ZONE, 16)])

    @pl.when(cid == 1)
    def _():
        pltpu.sync_copy(aggr_sh.at[pl.ds(zbase, _ZONE)],
                        p1_hbm.at[pl.ds(zbase, _ZONE)])

        @pl.when(sid == _NS - 1)
        def _():
            pltpu.sync_copy(aggr_sh.at[pl.ds(_NS * _ZONE, 16)],
                            p1_hbm.at[pl.ds(_NS * _ZONE, 16)])


_sc_mesh = plsc.VectorSubcoreMesh(core_axis_name="c", subcore_axis_name="s",
                                  num_cores=_NC, num_subcores=_NS)

_sc_edge = functools.partial(
    pl.kernel, _sc_edge_body,
    out_type=[
        jax.ShapeDtypeStruct((N, C), jnp.float32),
        jax.ShapeDtypeStruct((N, C), jnp.float32),
    ],
    mesh=_sc_mesh,
    scratch_types=[
        pltpu.VMEM((_CH,), jnp.int32),
        pltpu.VMEM((_CH,), jnp.int32),
        pltpu.VMEM((_CH, 2 * C), jnp.float32),
        pltpu.VMEM((_CH, 2 * C), jnp.float32),
        pltpu.VMEM((_CH, 2 * C), jnp.float32),
        pltpu.VMEM((_CH, C), jnp.float32),
        pltpu.VMEM_SHARED((N, C), jnp.float32),
        pltpu.SemaphoreType.DMA,
    ],
)


def kernel(x, node_batch, edge_index, edge_attr, Wf, bf, Ws, bs, W1, b1,
           bn_w, bn_b, W2, b2, ln_w, ln_b):
    L = Wf.shape[0]
    dst1 = edge_index[1].astype(jnp.int32)
    src1 = edge_index[0].astype(jnp.int32)
    nb3 = node_batch.astype(jnp.int32).reshape(_NB, 1, _BN)

    we_all = jnp.concatenate(
        [jnp.concatenate([Wf[l][2 * C:], Ws[l][2 * C:]], axis=1)
         for l in range(L)], axis=1)
    be_all = jnp.concatenate(
        [jnp.concatenate([bf[l], bs[l]]) for l in range(L)]).reshape(1, 3 * 2 * C)

    te = _edge_proj(edge_attr, we_all, be_all)

    sc_edge = _sc_edge()

    for l in range(L):
        wd = jnp.concatenate([Wf[l][:C], Ws[l][:C]], axis=1)
        wsr = jnp.concatenate([Wf[l][C:2 * C], Ws[l][C:2 * C]], axis=1)
        tg, ts = _node_proj(x, wd, wsr)
        p0, p1 = sc_edge(tg, ts, te[l], dst1, src1)
        x1, m, s = _stats(x, p0, p1)
        x2, gst = _mlp(x1, m, s, W1[l], b1[l].reshape(1, H),
                       bn_w[l].reshape(1, H), bn_b[l].reshape(1, H),
                       W2[l], b2[l].reshape(1, C), nb3)
        x = _ln_apply(x2, gst, nb3, ln_w[l].reshape(1, C),
                      ln_b[l].reshape(1, C))
    return x


# speedup vs baseline: 1.2131x; 1.2131x over previous
import functools

import jax
import jax.numpy as jnp
from jax import lax
from jax.experimental import pallas as pl
from jax.experimental.pallas import tpu as pltpu
from jax.experimental.pallas import tpu_sc as plsc

N = 10000
C = 128
D = 16
H = 4 * C
E = 320000
G = 16
EPS = 1e-5

_LP = (3.50755205e-06, 9.99792436e-01, -4.96977911e-01, 3.14590535e-01,
       -1.88782674e-01, 8.17268084e-02, -1.72080611e-02)

_NC = 2
_NS = 16
_NW = _NC * _NS
_CH = 16
_EPW = E // _NW
_CPW = _EPW // _CH
_IDXROWS = -(-_EPW // 128)
_ZONE = 624


_NB = 10
_BN = N // _NB
_EB = 160
_BE = E // _EB


def _node_proj_body(x_ref, wd_ref, ws_ref, tg_ref, ts_ref):
    xb = x_ref[...]
    tg_ref[...] = jnp.dot(xb, wd_ref[...], preferred_element_type=jnp.float32)
    ts_ref[...] = jnp.dot(xb, ws_ref[...], preferred_element_type=jnp.float32)


def _node_proj(x, wd, ws):
    return pl.pallas_call(
        _node_proj_body,
        grid=(_NB,),
        in_specs=[
            pl.BlockSpec((_BN, C), lambda i: (i, 0)),
            pl.BlockSpec((C, 2 * C), lambda i: (0, 0)),
            pl.BlockSpec((C, 2 * C), lambda i: (0, 0)),
        ],
        out_specs=[
            pl.BlockSpec((_BN, 2 * C), lambda i: (i, 0)),
            pl.BlockSpec((_BN, 2 * C), lambda i: (i, 0)),
        ],
        out_shape=[
            jax.ShapeDtypeStruct((N, 2 * C), jnp.float32),
            jax.ShapeDtypeStruct((N, 2 * C), jnp.float32),
        ],
    )(x, wd, ws)


def _edge_proj_body(ea_ref, we_ref, be_ref, t0_ref, t1_ref, t2_ref):
    eb = ea_ref[...]
    t = jnp.dot(eb, we_ref[...], preferred_element_type=jnp.float32) + be_ref[...]
    t0_ref[...] = t[:, 0:256]
    t1_ref[...] = t[:, 256:512]
    t2_ref[...] = t[:, 512:768]


def _edge_proj(edge_attr, we_all, be_all):
    return pl.pallas_call(
        _edge_proj_body,
        grid=(_EB,),
        in_specs=[
            pl.BlockSpec((_BE, D), lambda i: (i, 0)),
            pl.BlockSpec((D, 768), lambda i: (0, 0)),
            pl.BlockSpec((1, 768), lambda i: (0, 0)),
        ],
        out_specs=[pl.BlockSpec((_BE, 256), lambda i: (i, 0))] * 3,
        out_shape=[jax.ShapeDtypeStruct((E, 256), jnp.float32)] * 3,
    )(edge_attr, we_all, be_all)


def _stats_body(x_ref, p0_ref, p1_ref, x1_ref, m_ref, s_ref):
    i = pl.program_id(0)
    x1 = x_ref[...] + p0_ref[...] + p1_ref[...]
    x1_ref[...] = x1
    mb = jnp.sum(x1, axis=0, keepdims=True)
    sb = lax.dot_general(x1, x1, (((0,), (0,)), ((), ())),
                         preferred_element_type=jnp.float32)

    @pl.when(i == 0)
    def _():
        m_ref[...] = mb
        s_ref[...] = sb

    @pl.when(i > 0)
    def _():
        m_ref[...] += mb
        s_ref[...] += sb


def _stats(x, p0, p1):
    return pl.pallas_call(
        _stats_body,
        grid=(_NB,),
        in_specs=[pl.BlockSpec((_BN, C), lambda i: (i, 0))] * 3,
        out_specs=[
            pl.BlockSpec((_BN, C), lambda i: (i, 0)),
            pl.BlockSpec((1, C), lambda i: (0, 0)),
            pl.BlockSpec((C, C), lambda i: (0, 0)),
        ],
        out_shape=[
            jax.ShapeDtypeStruct((N, C), jnp.float32),
            jax.ShapeDtypeStruct((1, C), jnp.float32),
            jax.ShapeDtypeStruct((C, C), jnp.float32),
        ],
    )(x, p0, p1)


def _mlp_body(x1_ref, m_ref, s_ref, w1_ref, b1_ref, bnw_ref, bnb_ref,
              w2_ref, b2_ref, nb_ref, x2_ref, gst_ref, scale_v, shift_v):
    i = pl.program_id(0)

    @pl.when(i == 0)
    def _():
        w1 = w1_ref[...]
        mw = jnp.dot(m_ref[...] / N, w1, preferred_element_type=jnp.float32)
        sw = jnp.dot(s_ref[...], w1, preferred_element_type=jnp.float32)
        q = jnp.sum(w1 * sw, axis=0, keepdims=True)
        b1 = b1_ref[...]
        mu = mw + b1
        ex2 = q / N + 2.0 * b1 * mw + b1 * b1
        var = ex2 - mu * mu
        scale = bnw_ref[...] * lax.rsqrt(var + EPS)
        scale_v[...] = scale
        shift_v[...] = bnb_ref[...] - mw * scale

    x1 = x1_ref[...]
    heff = jnp.dot(x1, w1_ref[...], preferred_element_type=jnp.float32)
    hn = jnp.maximum(heff * scale_v[...] + shift_v[...], 0.0)
    x2 = x1 + jnp.dot(hn, w2_ref[...], preferred_element_type=jnp.float32) + b2_ref[...]
    x2_ref[...] = x2

    nb = nb_ref[0, 0, :]
    oh = (nb[:, None] == lax.broadcasted_iota(jnp.int32, (1, C), 1)
          ).astype(jnp.float32)
    s1 = jnp.dot(jnp.sum(x2, axis=1)[None, :], oh,
                 preferred_element_type=jnp.float32)
    s2 = jnp.dot(jnp.sum(x2 * x2, axis=1)[None, :], oh,
                 preferred_element_type=jnp.float32)
    dg = jnp.sum(oh, axis=0, keepdims=True)
    row = jnp.concatenate([s1, s2, dg], axis=0)

    @pl.when(i == 0)
    def _():
        gst_ref[...] = row

    @pl.when(i > 0)
    def _():
        gst_ref[...] += row


def _mlp(x1, m, s, w1, b1, bnw, bnb, w2, b2, nb3):
    return pl.pallas_call(
        _mlp_body,
        grid=(_NB,),
        in_specs=[
            pl.BlockSpec((_BN, C), lambda i: (i, 0)),
            pl.BlockSpec((1, C), lambda i: (0, 0)),
            pl.BlockSpec((C, C), lambda i: (0, 0)),
            pl.BlockSpec((C, H), lambda i: (0, 0)),
            pl.BlockSpec((1, H), lambda i: (0, 0)),
            pl.BlockSpec((1, H), lambda i: (0, 0)),
            pl.BlockSpec((1, H), lambda i: (0, 0)),
            pl.BlockSpec((H, C), lambda i: (0, 0)),
            pl.BlockSpec((1, C), lambda i: (0, 0)),
            pl.BlockSpec((1, 1, _BN), lambda i: (i, 0, 0)),
        ],
        out_specs=[
            pl.BlockSpec((_BN, C), lambda i: (i, 0)),
            pl.BlockSpec((3, C), lambda i: (0, 0)),
        ],
        out_shape=[
            jax.ShapeDtypeStruct((N, C), jnp.float32),
            jax.ShapeDtypeStruct((3, C), jnp.float32),
        ],
        scratch_shapes=[
            pltpu.VMEM((1, H), jnp.float32),
            pltpu.VMEM((1, H), jnp.float32),
        ],
    )(x1, m, s, w1, b1, bnw, bnb, w2, b2, nb3)


def _ln_body(x2_ref, gst_ref, nb_ref, lnw_ref, lnb_ref, out_ref):
    s1 = gst_ref[0:1, :]
    s2 = gst_ref[1:2, :]
    dg = gst_ref[2:3, :]
    norm = jnp.maximum(dg, 1.0) * C
    mean = s1 / norm
    var = (s2 - 2.0 * mean * s1 + mean * mean * dg * C) / norm
    inv = lax.rsqrt(var + EPS)

    nb = nb_ref[0, 0, :]
    oh = (nb[:, None] == lax.broadcasted_iota(jnp.int32, (1, C), 1)
          ).astype(jnp.float32)
    mean_n = jnp.sum(oh * mean, axis=1, keepdims=True)
    inv_n = jnp.sum(oh * inv, axis=1, keepdims=True)
    x2 = x2_ref[...]
    out_ref[...] = (x2 - mean_n) * inv_n * lnw_ref[...] + lnb_ref[...]


def _ln_apply(x2, gst, nb3, lnw, lnb):
    return pl.pallas_call(
        _ln_body,
        grid=(_NB,),
        in_specs=[
            pl.BlockSpec((_BN, C), lambda i: (i, 0)),
            pl.BlockSpec((3, C), lambda i: (0, 0)),
            pl.BlockSpec((1, 1, _BN), lambda i: (i, 0, 0)),
            pl.BlockSpec((1, C), lambda i: (0, 0)),
            pl.BlockSpec((1, C), lambda i: (0, 0)),
        ],
        out_specs=pl.BlockSpec((_BN, C), lambda i: (i, 0)),
        out_shape=jax.ShapeDtypeStruct((N, C), jnp.float32),
    )(x2, gst, nb3, lnw, lnb)


def _sc_edge_body(tg_hbm, ts_hbm, te_hbm, dst_hbm, src_hbm,
                  p0_hbm, p1_hbm,
                  dsti_v, srci_v, dg0, dg1, sg0, sg1, ds0, ds1,
                  g0, g1, s0, s1, t0, t1, m0, m1,
                  aggr_sh, semL, semS):
    dg_v = (dg0, dg1)
    sg_v = (sg0, sg1)
    ds_v = (ds0, ds1)
    g_v = (g0, g1)
    s_v = (s0, s1)
    t_v = (t0, t1)
    m_v = (m0, m1)
    cid = lax.axis_index("c")
    sid = lax.axis_index("s")
    wid = cid * _NS + sid
    cbase = wid * _CPW

    def _fill(buf, big, j):
        buf[...] = big[j >> 3, pl.ds((j & 7) * 16, 16)]

    def _zrow(r, _):
        for jj in range(C // 16):
            m_v[0][r, pl.ds(jj * 16, 16)] = jnp.zeros((16,), jnp.float32)
        return _

    lax.fori_loop(0, _CH, _zrow, None)
    zbase = sid * _ZONE
    for k in range(_ZONE // _CH):
        pltpu.sync_copy(m_v[0], aggr_sh.at[pl.ds(zbase + k * _CH, _CH)])

    @pl.when(sid == _NS - 1)
    def _():
        pltpu.sync_copy(m_v[0], aggr_sh.at[pl.ds(_NS * _ZONE, 16)])

    pltpu.sync_copy(dst_hbm.at[wid], dsti_v)
    pltpu.sync_copy(src_hbm.at[wid], srci_v)
    plsc.subcore_barrier()

    def _start_loads(j, b):
        pltpu.async_copy(tg_hbm.at[dg_v[b]], g_v[b], semL)
        pltpu.async_copy(ts_hbm.at[sg_v[b]], s_v[b], semL)
        pltpu.async_copy(te_hbm.at[pl.ds((cbase + j) * _CH, _CH)], t_v[b], semL)

    def _drain_loads(b):
        pltpu.make_async_copy(tg_hbm.at[pl.ds(0, _CH)], g_v[b], semL).wait()
        pltpu.make_async_copy(ts_hbm.at[pl.ds(0, _CH)], s_v[b], semL).wait()
        pltpu.make_async_copy(te_hbm.at[pl.ds(0, _CH)], t_v[b], semL).wait()

    def _compute(b):
        def _edge(e, _2):
            for jj in range(C // 16):
                o = jj * 16
                g = (g_v[b][e, pl.ds(o, 16)] + s_v[b][e, pl.ds(o, 16)]
                     + t_v[b][e, pl.ds(o, 16)])
                s = (g_v[b][e, pl.ds(C + o, 16)] + s_v[b][e, pl.ds(C + o, 16)]
                     + t_v[b][e, pl.ds(C + o, 16)])
                sig = 1.0 / (1.0 + jnp.exp(-g))
                t = jnp.exp(-jnp.abs(s))
                p = jnp.full((16,), _LP[6], jnp.float32)
                for cf in (_LP[5], _LP[4], _LP[3], _LP[2], _LP[1], _LP[0]):
                    p = p * t + cf
                sp = jnp.maximum(s, 0.0) + p
                m_v[b][e, pl.ds(o, 16)] = sig * sp
            return _2

        lax.fori_loop(0, _CH, _edge, None)

    def _chunk(j, b):
        _drain_loads(b)

        @pl.when(j >= 2)
        def _():
            pltpu.make_async_copy(p0_hbm.at[pl.ds(0, _CH)], m_v[b], semS).wait()

        _fill(ds_v[b], dsti_v, j)
        _compute(b)

        @pl.when(j + 2 < _CPW)
        def _():
            _fill(dg_v[b], dsti_v, j + 2)
            _fill(sg_v[b], srci_v, j + 2)
            _start_loads(j + 2, b)

        pltpu.async_copy(m_v[b], aggr_sh.at[ds_v[b]], semS, add=True)

    _fill(dg0, dsti_v, 0)
    _fill(sg0, srci_v, 0)
    _start_loads(0, 0)
    _fill(dg1, dsti_v, 1)
    _fill(sg1, srci_v, 1)
    _start_loads(1, 1)

    def _pair(g, _):
        _chunk(2 * g, 0)
        _chunk(2 * g + 1, 1)
        return _

    lax.fori_loop(0, _CPW // 2, _pair, None)
    if _CPW % 2:
        _chunk(_CPW - 1, 0)
    pltpu.make_async_copy(p0_hbm.at[pl.ds(0, _CH)], m_v[0], semS).wait()
    pltpu.make_async_copy(p0_hbm.at[pl.ds(0, _CH)], m_v[1], semS).wait()
    plsc.subcore_barrier()

    @pl.when(cid == 0)
    def _():
        pltpu.sync_copy(aggr_sh.at[pl.ds(zbase, _ZONE)],
                        p0_hbm.at[pl.ds(zbase, _ZONE)])

        @pl.when(sid == _NS - 1)
        def _():
            pltpu.sync_copy(aggr_sh.at[pl.ds(_NS * _ZONE, 16)],
                            p0_hbm.at[pl.ds(_NS * _ZONE, 16)])

    @pl.when(cid == 1)
    def _():
        pltpu.sync_copy(aggr_sh.at[pl.ds(zbase, _ZONE)],
                        p1_hbm.at[pl.ds(zbase, _ZONE)])

        @pl.when(sid == _NS - 1)
        def _():
            pltpu.sync_copy(aggr_sh.at[pl.ds(_NS * _ZONE, 16)],
                            p1_hbm.at[pl.ds(_NS * _ZONE, 16)])


_sc_mesh = plsc.VectorSubcoreMesh(core_axis_name="c", subcore_axis_name="s",
                                  num_cores=_NC, num_subcores=_NS)

_sc_edge = functools.partial(
    pl.kernel, _sc_edge_body,
    out_type=[
        jax.ShapeDtypeStruct((N, C), jnp.float32),
        jax.ShapeDtypeStruct((N, C), jnp.float32),
    ],
    mesh=_sc_mesh,
    scratch_types=(
        [pltpu.VMEM((_IDXROWS, 128), jnp.int32)] * 2
        + [pltpu.VMEM((_CH,), jnp.int32)] * 6
        + [pltpu.VMEM((_CH, 2 * C), jnp.float32)] * 6
        + [pltpu.VMEM((_CH, C), jnp.float32)] * 2
        + [pltpu.VMEM_SHARED((N, C), jnp.float32),
           pltpu.SemaphoreType.DMA, pltpu.SemaphoreType.DMA]
    ),
)


def kernel(x, node_batch, edge_index, edge_attr, Wf, bf, Ws, bs, W1, b1,
           bn_w, bn_b, W2, b2, ln_w, ln_b):
    L = Wf.shape[0]
    pad = _IDXROWS * 128 - _EPW
    dst3 = jnp.pad(edge_index[1].astype(jnp.int32).reshape(_NW, _EPW),
                   ((0, 0), (0, pad))).reshape(_NW, _IDXROWS, 128)
    src3 = jnp.pad(edge_index[0].astype(jnp.int32).reshape(_NW, _EPW),
                   ((0, 0), (0, pad))).reshape(_NW, _IDXROWS, 128)
    nb3 = node_batch.astype(jnp.int32).reshape(_NB, 1, _BN)

    we_all = jnp.concatenate(
        [jnp.concatenate([Wf[l][2 * C:], Ws[l][2 * C:]], axis=1)
         for l in range(L)], axis=1)
    be_all = jnp.concatenate(
        [jnp.concatenate([bf[l], bs[l]]) for l in range(L)]).reshape(1, 3 * 2 * C)

    te = _edge_proj(edge_attr, we_all, be_all)

    sc_edge = _sc_edge()

    for l in range(L):
        wd = jnp.concatenate([Wf[l][:C], Ws[l][:C]], axis=1)
        wsr = jnp.concatenate([Wf[l][C:2 * C], Ws[l][C:2 * C]], axis=1)
        tg, ts = _node_proj(x, wd, wsr)
        p0, p1 = sc_edge(tg, ts, te[l], dst3, src3)
        x1, m, s = _stats(x, p0, p1)
        x2, gst = _mlp(x1, m, s, W1[l], b1[l].reshape(1, H),
                       bn_w[l].reshape(1, H), bn_b[l].reshape(1, H),
                       W2[l], b2[l].reshape(1, C), nb3)
        x = _ln_apply(x2, gst, nb3, ln_w[l].reshape(1, C),
                      ln_b[l].reshape(1, C))
    return x


# speedup vs baseline: 1.3108x; 1.0806x over previous
import functools

import jax
import jax.numpy as jnp
from jax import lax
from jax.experimental import pallas as pl
from jax.experimental.pallas import tpu as pltpu
from jax.experimental.pallas import tpu_sc as plsc

N = 10000
C = 128
D = 16
H = 4 * C
E = 320000
G = 16
EPS = 1e-5

_LP = (2.21170312e-05, 9.99010447e-01, -4.89156847e-01, 2.83304325e-01,
       -1.30119415e-01, 3.01026250e-02)

_NC = 2
_NS = 16
_NW = _NC * _NS
_CH = 16
_EPW = E // _NW
_CPW = _EPW // _CH
_IDXROWS = -(-_EPW // 128)
_ZONE = 624


_NB = 10
_BN = N // _NB
_EB = 160
_BE = E // _EB


def _node_proj_body(x_ref, wd_ref, ws_ref, tg_ref, ts_ref):
    xb = x_ref[...]
    tg_ref[...] = jnp.dot(xb, wd_ref[...], preferred_element_type=jnp.float32)
    ts_ref[...] = jnp.dot(xb, ws_ref[...], preferred_element_type=jnp.float32)


def _node_proj(x, wd, ws):
    return pl.pallas_call(
        _node_proj_body,
        grid=(_NB,),
        in_specs=[
            pl.BlockSpec((_BN, C), lambda i: (i, 0)),
            pl.BlockSpec((C, 2 * C), lambda i: (0, 0)),
            pl.BlockSpec((C, 2 * C), lambda i: (0, 0)),
        ],
        out_specs=[
            pl.BlockSpec((_BN, 2 * C), lambda i: (i, 0)),
            pl.BlockSpec((_BN, 2 * C), lambda i: (i, 0)),
        ],
        out_shape=[
            jax.ShapeDtypeStruct((N, 2 * C), jnp.float32),
            jax.ShapeDtypeStruct((N, 2 * C), jnp.float32),
        ],
    )(x, wd, ws)


def _edge_proj_body(ea_ref, we_ref, be_ref, t0_ref, t1_ref, t2_ref):
    eb = ea_ref[...]
    t = jnp.dot(eb, we_ref[...], preferred_element_type=jnp.float32) + be_ref[...]
    t0_ref[...] = t[:, 0:256]
    t1_ref[...] = t[:, 256:512]
    t2_ref[...] = t[:, 512:768]


def _edge_proj(edge_attr, we_all, be_all):
    return pl.pallas_call(
        _edge_proj_body,
        grid=(_EB,),
        in_specs=[
            pl.BlockSpec((_BE, D), lambda i: (i, 0)),
            pl.BlockSpec((D, 768), lambda i: (0, 0)),
            pl.BlockSpec((1, 768), lambda i: (0, 0)),
        ],
        out_specs=[pl.BlockSpec((_BE, 256), lambda i: (i, 0))] * 3,
        out_shape=[jax.ShapeDtypeStruct((E, 256), jnp.float32)] * 3,
    )(edge_attr, we_all, be_all)


def _stats_body(x_ref, p0_ref, p1_ref, x1_ref, m_ref, s_ref):
    i = pl.program_id(0)
    x1 = x_ref[...] + p0_ref[...] + p1_ref[...]
    x1_ref[...] = x1
    mb = jnp.sum(x1, axis=0, keepdims=True)
    sb = lax.dot_general(x1, x1, (((0,), (0,)), ((), ())),
                         preferred_element_type=jnp.float32)

    @pl.when(i == 0)
    def _():
        m_ref[...] = mb
        s_ref[...] = sb

    @pl.when(i > 0)
    def _():
        m_ref[...] += mb
        s_ref[...] += sb


def _stats(x, p0, p1):
    return pl.pallas_call(
        _stats_body,
        grid=(_NB,),
        in_specs=[pl.BlockSpec((_BN, C), lambda i: (i, 0))] * 3,
        out_specs=[
            pl.BlockSpec((_BN, C), lambda i: (i, 0)),
            pl.BlockSpec((1, C), lambda i: (0, 0)),
            pl.BlockSpec((C, C), lambda i: (0, 0)),
        ],
        out_shape=[
            jax.ShapeDtypeStruct((N, C), jnp.float32),
            jax.ShapeDtypeStruct((1, C), jnp.float32),
            jax.ShapeDtypeStruct((C, C), jnp.float32),
        ],
    )(x, p0, p1)


def _mlp_body(x1_ref, m_ref, s_ref, w1_ref, b1_ref, bnw_ref, bnb_ref,
              w2_ref, b2_ref, nb_ref, x2_ref, gst_ref, scale_v, shift_v):
    i = pl.program_id(0)

    @pl.when(i == 0)
    def _():
        w1 = w1_ref[...]
        mw = jnp.dot(m_ref[...] / N, w1, preferred_element_type=jnp.float32)
        sw = jnp.dot(s_ref[...], w1, preferred_element_type=jnp.float32)
        q = jnp.sum(w1 * sw, axis=0, keepdims=True)
        b1 = b1_ref[...]
        mu = mw + b1
        ex2 = q / N + 2.0 * b1 * mw + b1 * b1
        var = ex2 - mu * mu
        scale = bnw_ref[...] * lax.rsqrt(var + EPS)
        scale_v[...] = scale
        shift_v[...] = bnb_ref[...] - mw * scale

    x1 = x1_ref[...]
    heff = jnp.dot(x1, w1_ref[...], preferred_element_type=jnp.float32)
    hn = jnp.maximum(heff * scale_v[...] + shift_v[...], 0.0)
    x2 = x1 + jnp.dot(hn, w2_ref[...], preferred_element_type=jnp.float32) + b2_ref[...]
    x2_ref[...] = x2

    nb = nb_ref[0, 0, :]
    oh = (nb[:, None] == lax.broadcasted_iota(jnp.int32, (1, C), 1)
          ).astype(jnp.float32)
    s1 = jnp.dot(jnp.sum(x2, axis=1)[None, :], oh,
                 preferred_element_type=jnp.float32)
    s2 = jnp.dot(jnp.sum(x2 * x2, axis=1)[None, :], oh,
                 preferred_element_type=jnp.float32)
    dg = jnp.sum(oh, axis=0, keepdims=True)
    row = jnp.concatenate([s1, s2, dg], axis=0)

    @pl.when(i == 0)
    def _():
        gst_ref[...] = row

    @pl.when(i > 0)
    def _():
        gst_ref[...] += row


def _mlp(x1, m, s, w1, b1, bnw, bnb, w2, b2, nb3):
    return pl.pallas_call(
        _mlp_body,
        grid=(_NB,),
        in_specs=[
            pl.BlockSpec((_BN, C), lambda i: (i, 0)),
            pl.BlockSpec((1, C), lambda i: (0, 0)),
            pl.BlockSpec((C, C), lambda i: (0, 0)),
            pl.BlockSpec((C, H), lambda i: (0, 0)),
            pl.BlockSpec((1, H), lambda i: (0, 0)),
            pl.BlockSpec((1, H), lambda i: (0, 0)),
            pl.BlockSpec((1, H), lambda i: (0, 0)),
            pl.BlockSpec((H, C), lambda i: (0, 0)),
            pl.BlockSpec((1, C), lambda i: (0, 0)),
            pl.BlockSpec((1, 1, _BN), lambda i: (i, 0, 0)),
        ],
        out_specs=[
            pl.BlockSpec((_BN, C), lambda i: (i, 0)),
            pl.BlockSpec((3, C), lambda i: (0, 0)),
        ],
        out_shape=[
            jax.ShapeDtypeStruct((N, C), jnp.float32),
            jax.ShapeDtypeStruct((3, C), jnp.float32),
        ],
        scratch_shapes=[
            pltpu.VMEM((1, H), jnp.float32),
            pltpu.VMEM((1, H), jnp.float32),
        ],
    )(x1, m, s, w1, b1, bnw, bnb, w2, b2, nb3)


def _ln_body(x2_ref, gst_ref, nb_ref, lnw_ref, lnb_ref, out_ref):
    s1 = gst_ref[0:1, :]
    s2 = gst_ref[1:2, :]
    dg = gst_ref[2:3, :]
    norm = jnp.maximum(dg, 1.0) * C
    mean = s1 / norm
    var = (s2 - 2.0 * mean * s1 + mean * mean * dg * C) / norm
    inv = lax.rsqrt(var + EPS)

    nb = nb_ref[0, 0, :]
    oh = (nb[:, None] == lax.broadcasted_iota(jnp.int32, (1, C), 1)
          ).astype(jnp.float32)
    mean_n = jnp.sum(oh * mean, axis=1, keepdims=True)
    inv_n = jnp.sum(oh * inv, axis=1, keepdims=True)
    x2 = x2_ref[...]
    out_ref[...] = (x2 - mean_n) * inv_n * lnw_ref[...] + lnb_ref[...]


def _ln_apply(x2, gst, nb3, lnw, lnb):
    return pl.pallas_call(
        _ln_body,
        grid=(_NB,),
        in_specs=[
            pl.BlockSpec((_BN, C), lambda i: (i, 0)),
            pl.BlockSpec((3, C), lambda i: (0, 0)),
            pl.BlockSpec((1, 1, _BN), lambda i: (i, 0, 0)),
            pl.BlockSpec((1, C), lambda i: (0, 0)),
            pl.BlockSpec((1, C), lambda i: (0, 0)),
        ],
        out_specs=pl.BlockSpec((_BN, C), lambda i: (i, 0)),
        out_shape=jax.ShapeDtypeStruct((N, C), jnp.float32),
    )(x2, gst, nb3, lnw, lnb)


def _sc_edge_body(tg_hbm, ts_hbm, te_hbm, dst_hbm, src_hbm,
                  p0_hbm, p1_hbm,
                  dsti_v, srci_v, dg0, dg1, sg0, sg1, ds0, ds1,
                  g0, g1, s0, s1, t0, t1, m0, m1,
                  aggr_sh, semL, semS):
    dg_v = (dg0, dg1)
    sg_v = (sg0, sg1)
    ds_v = (ds0, ds1)
    g_v = (g0, g1)
    s_v = (s0, s1)
    t_v = (t0, t1)
    m_v = (m0, m1)
    cid = lax.axis_index("c")
    sid = lax.axis_index("s")
    wid = cid * _NS + sid
    cbase = wid * _CPW

    def _fill(buf, big, j):
        buf[...] = big[j >> 3, pl.ds((j & 7) * 16, 16)]

    def _zrow(r, _):
        for jj in range(C // 16):
            m_v[0][r, pl.ds(jj * 16, 16)] = jnp.zeros((16,), jnp.float32)
        return _

    lax.fori_loop(0, _CH, _zrow, None)
    zbase = sid * _ZONE
    for k in range(_ZONE // _CH):
        pltpu.sync_copy(m_v[0], aggr_sh.at[pl.ds(zbase + k * _CH, _CH)])

    @pl.when(sid == _NS - 1)
    def _():
        pltpu.sync_copy(m_v[0], aggr_sh.at[pl.ds(_NS * _ZONE, 16)])

    pltpu.sync_copy(dst_hbm.at[wid], dsti_v)
    pltpu.sync_copy(src_hbm.at[wid], srci_v)
    plsc.subcore_barrier()

    def _start_loads(j, b):
        pltpu.async_copy(tg_hbm.at[dg_v[b]], g_v[b], semL)
        pltpu.async_copy(ts_hbm.at[sg_v[b]], s_v[b], semL)
        pltpu.async_copy(te_hbm.at[pl.ds((cbase + j) * _CH, _CH)], t_v[b], semL)

    def _drain_loads(b):
        pltpu.make_async_copy(tg_hbm.at[pl.ds(0, _CH)], g_v[b], semL).wait()
        pltpu.make_async_copy(ts_hbm.at[pl.ds(0, _CH)], s_v[b], semL).wait()
        pltpu.make_async_copy(te_hbm.at[pl.ds(0, _CH)], t_v[b], semL).wait()

    def _compute(b):
        @plsc.parallel_loop(0, _CH, 1, unroll=4)
        def _edge(e):
            for jj in range(C // 16):
                o = jj * 16
                g = (g_v[b][e, pl.ds(o, 16)] + s_v[b][e, pl.ds(o, 16)]
                     + t_v[b][e, pl.ds(o, 16)])
                s = (g_v[b][e, pl.ds(C + o, 16)] + s_v[b][e, pl.ds(C + o, 16)]
                     + t_v[b][e, pl.ds(C + o, 16)])
                sig = 1.0 / (1.0 + jnp.exp(-g))
                t = jnp.exp(-jnp.abs(s))
                p = t * _LP[5] + _LP[4]
                for cf in (_LP[3], _LP[2], _LP[1], _LP[0]):
                    p = p * t + cf
                sp = jnp.maximum(s, 0.0) + p
                m_v[b][e, pl.ds(o, 16)] = sig * sp

    def _chunk(j, b):
        _drain_loads(b)

        @pl.when(j >= 2)
        def _():
            pltpu.make_async_copy(p0_hbm.at[pl.ds(0, _CH)], m_v[b], semS).wait()

        _fill(ds_v[b], dsti_v, j)
        _compute(b)

        @pl.when(j + 2 < _CPW)
        def _():
            _fill(dg_v[b], dsti_v, j + 2)
            _fill(sg_v[b], srci_v, j + 2)
            _start_loads(j + 2, b)

        pltpu.async_copy(m_v[b], aggr_sh.at[ds_v[b]], semS, add=True)

    _fill(dg0, dsti_v, 0)
    _fill(sg0, srci_v, 0)
    _start_loads(0, 0)
    _fill(dg1, dsti_v, 1)
    _fill(sg1, srci_v, 1)
    _start_loads(1, 1)

    def _pair(g, _):
        _chunk(2 * g, 0)
        _chunk(2 * g + 1, 1)
        return _

    lax.fori_loop(0, _CPW // 2, _pair, None)
    if _CPW % 2:
        _chunk(_CPW - 1, 0)
    pltpu.make_async_copy(p0_hbm.at[pl.ds(0, _CH)], m_v[0], semS).wait()
    pltpu.make_async_copy(p0_hbm.at[pl.ds(0, _CH)], m_v[1], semS).wait()
    plsc.subcore_barrier()

    @pl.when(cid == 0)
    def _():
        pltpu.sync_copy(aggr_sh.at[pl.ds(zbase, _ZONE)],
                        p0_hbm.at[pl.ds(zbase, _ZONE)])

        @pl.when(sid == _NS - 1)
        def _():
            pltpu.sync_copy(aggr_sh.at[pl.ds(_NS * _ZONE, 16)],
                            p0_hbm.at[pl.ds(_NS * _ZONE, 16)])

    @pl.when(cid == 1)
    def _():
        pltpu.sync_copy(aggr_sh.at[pl.ds(zbase, _ZONE)],
                        p1_hbm.at[pl.ds(zbase, _ZONE)])

        @pl.when(sid == _NS - 1)
        def _():
            pltpu.sync_copy(aggr_sh.at[pl.ds(_NS * _ZONE, 16)],
                            p1_hbm.at[pl.ds(_NS * _ZONE, 16)])


_sc_mesh = plsc.VectorSubcoreMesh(core_axis_name="c", subcore_axis_name="s",
                                  num_cores=_NC, num_subcores=_NS)

_sc_edge = functools.partial(
    pl.kernel, _sc_edge_body,
    out_type=[
        jax.ShapeDtypeStruct((N, C), jnp.float32),
        jax.ShapeDtypeStruct((N, C), jnp.float32),
    ],
    mesh=_sc_mesh,
    scratch_types=(
        [pltpu.VMEM((_IDXROWS, 128), jnp.int32)] * 2
        + [pltpu.VMEM((_CH,), jnp.int32)] * 6
        + [pltpu.VMEM((_CH, 2 * C), jnp.float32)] * 6
        + [pltpu.VMEM((_CH, C), jnp.float32)] * 2
        + [pltpu.VMEM_SHARED((N, C), jnp.float32),
           pltpu.SemaphoreType.DMA, pltpu.SemaphoreType.DMA]
    ),
)


def kernel(x, node_batch, edge_index, edge_attr, Wf, bf, Ws, bs, W1, b1,
           bn_w, bn_b, W2, b2, ln_w, ln_b):
    L = Wf.shape[0]
    pad = _IDXROWS * 128 - _EPW
    dst3 = jnp.pad(edge_index[1].astype(jnp.int32).reshape(_NW, _EPW),
                   ((0, 0), (0, pad))).reshape(_NW, _IDXROWS, 128)
    src3 = jnp.pad(edge_index[0].astype(jnp.int32).reshape(_NW, _EPW),
                   ((0, 0), (0, pad))).reshape(_NW, _IDXROWS, 128)
    nb3 = node_batch.astype(jnp.int32).reshape(_NB, 1, _BN)

    we_all = jnp.concatenate(
        [jnp.concatenate([Wf[l][2 * C:], Ws[l][2 * C:]], axis=1)
         for l in range(L)], axis=1)
    be_all = jnp.concatenate(
        [jnp.concatenate([bf[l], bs[l]]) for l in range(L)]).reshape(1, 3 * 2 * C)

    te = _edge_proj(edge_attr, we_all, be_all)

    sc_edge = _sc_edge()

    for l in range(L):
        wd = jnp.concatenate([Wf[l][:C], Ws[l][:C]], axis=1)
        wsr = jnp.concatenate([Wf[l][C:2 * C], Ws[l][C:2 * C]], axis=1)
        tg, ts = _node_proj(x, wd, wsr)
        p0, p1 = sc_edge(tg, ts, te[l], dst3, src3)
        x1, m, s = _stats(x, p0, p1)
        x2, gst = _mlp(x1, m, s, W1[l], b1[l].reshape(1, H),
                       bn_w[l].reshape(1, H), bn_b[l].reshape(1, H),
                       W2[l], b2[l].reshape(1, C), nb3)
        x = _ln_apply(x2, gst, nb3, ln_w[l].reshape(1, C),
                      ln_b[l].reshape(1, C))
    return x


# speedup vs baseline: 2.6564x; 2.0265x over previous
import functools

import jax
import jax.numpy as jnp
from jax import lax
from jax.experimental import pallas as pl
from jax.experimental.pallas import tpu as pltpu
from jax.experimental.pallas import tpu_sc as plsc

N = 10000
C = 128
D = 16
H = 4 * C
E = 320000
G = 16
EPS = 1e-5

_LP = (2.21170312e-05, 9.99010447e-01, -4.89156847e-01, 2.83304325e-01,
       -1.30119415e-01, 3.01026250e-02)

_NC = 2
_NS = 16
_NW = _NC * _NS
_CH = 16
_EPW = E // _NW
_CPW = _EPW // _CH
_IDXROWS = -(-_EPW // 128)
_ZONE = 624


_NB = 10
_BN = N // _NB
_EB = 160
_BE = E // _EB


def _node_proj_body(x_ref, wd_ref, ws_ref, tg_ref, ts_ref):
    xb = x_ref[...]
    tg_ref[...] = jnp.dot(xb, wd_ref[...], preferred_element_type=jnp.float32)
    ts_ref[...] = jnp.dot(xb, ws_ref[...], preferred_element_type=jnp.float32)


def _node_proj(x, wd, ws):
    return pl.pallas_call(
        _node_proj_body,
        grid=(_NB,),
        in_specs=[
            pl.BlockSpec((_BN, C), lambda i: (i, 0)),
            pl.BlockSpec((C, 2 * C), lambda i: (0, 0)),
            pl.BlockSpec((C, 2 * C), lambda i: (0, 0)),
        ],
        out_specs=[
            pl.BlockSpec((_BN, 2 * C), lambda i: (i, 0)),
            pl.BlockSpec((_BN, 2 * C), lambda i: (i, 0)),
        ],
        out_shape=[
            jax.ShapeDtypeStruct((N, 2 * C), jnp.float32),
            jax.ShapeDtypeStruct((N, 2 * C), jnp.float32),
        ],
    )(x, wd, ws)


def _edge_proj_body(ea_ref, we_ref, be_ref, t0_ref, t1_ref, t2_ref):
    eb = ea_ref[...]
    t = jnp.dot(eb, we_ref[...], preferred_element_type=jnp.float32) + be_ref[...]
    t0_ref[...] = t[:, 0:256]
    t1_ref[...] = t[:, 256:512]
    t2_ref[...] = t[:, 512:768]


def _edge_proj(edge_attr, we_all, be_all):
    return pl.pallas_call(
        _edge_proj_body,
        grid=(_EB,),
        in_specs=[
            pl.BlockSpec((_BE, D), lambda i: (i, 0)),
            pl.BlockSpec((D, 768), lambda i: (0, 0)),
            pl.BlockSpec((1, 768), lambda i: (0, 0)),
        ],
        out_specs=[pl.BlockSpec((_BE, 256), lambda i: (i, 0))] * 3,
        out_shape=[jax.ShapeDtypeStruct((E, 256), jnp.float32)] * 3,
    )(edge_attr, we_all, be_all)


def _stats_body(x_ref, p0_ref, p1_ref, x1_ref, m_ref, s_ref):
    i = pl.program_id(0)
    x1 = x_ref[...] + p0_ref[...] + p1_ref[...]
    x1_ref[...] = x1
    mb = jnp.sum(x1, axis=0, keepdims=True)
    sb = lax.dot_general(x1, x1, (((0,), (0,)), ((), ())),
                         preferred_element_type=jnp.float32)

    @pl.when(i == 0)
    def _():
        m_ref[...] = mb
        s_ref[...] = sb

    @pl.when(i > 0)
    def _():
        m_ref[...] += mb
        s_ref[...] += sb


def _stats(x, p0, p1):
    return pl.pallas_call(
        _stats_body,
        grid=(_NB,),
        in_specs=[pl.BlockSpec((_BN, C), lambda i: (i, 0))] * 3,
        out_specs=[
            pl.BlockSpec((_BN, C), lambda i: (i, 0)),
            pl.BlockSpec((1, C), lambda i: (0, 0)),
            pl.BlockSpec((C, C), lambda i: (0, 0)),
        ],
        out_shape=[
            jax.ShapeDtypeStruct((N, C), jnp.float32),
            jax.ShapeDtypeStruct((1, C), jnp.float32),
            jax.ShapeDtypeStruct((C, C), jnp.float32),
        ],
    )(x, p0, p1)


def _mlp_body(x1_ref, m_ref, s_ref, w1_ref, b1_ref, bnw_ref, bnb_ref,
              w2_ref, b2_ref, nb_ref, x2_ref, gst_ref, scale_v, shift_v):
    i = pl.program_id(0)

    @pl.when(i == 0)
    def _():
        w1 = w1_ref[...]
        mw = jnp.dot(m_ref[...] / N, w1, preferred_element_type=jnp.float32)
        sw = jnp.dot(s_ref[...], w1, preferred_element_type=jnp.float32)
        q = jnp.sum(w1 * sw, axis=0, keepdims=True)
        b1 = b1_ref[...]
        mu = mw + b1
        ex2 = q / N + 2.0 * b1 * mw + b1 * b1
        var = ex2 - mu * mu
        scale = bnw_ref[...] * lax.rsqrt(var + EPS)
        scale_v[...] = scale
        shift_v[...] = bnb_ref[...] - mw * scale

    x1 = x1_ref[...]
    heff = jnp.dot(x1, w1_ref[...], preferred_element_type=jnp.float32)
    hn = jnp.maximum(heff * scale_v[...] + shift_v[...], 0.0)
    x2 = x1 + jnp.dot(hn, w2_ref[...], preferred_element_type=jnp.float32) + b2_ref[...]
    x2_ref[...] = x2

    nb = nb_ref[0, 0, :]
    oh = (nb[:, None] == lax.broadcasted_iota(jnp.int32, (1, C), 1)
          ).astype(jnp.float32)
    s1 = jnp.dot(jnp.sum(x2, axis=1)[None, :], oh,
                 preferred_element_type=jnp.float32)
    s2 = jnp.dot(jnp.sum(x2 * x2, axis=1)[None, :], oh,
                 preferred_element_type=jnp.float32)
    dg = jnp.sum(oh, axis=0, keepdims=True)
    row = jnp.concatenate([s1, s2, dg], axis=0)

    @pl.when(i == 0)
    def _():
        gst_ref[...] = row

    @pl.when(i > 0)
    def _():
        gst_ref[...] += row


def _mlp(x1, m, s, w1, b1, bnw, bnb, w2, b2, nb3):
    return pl.pallas_call(
        _mlp_body,
        grid=(_NB,),
        in_specs=[
            pl.BlockSpec((_BN, C), lambda i: (i, 0)),
            pl.BlockSpec((1, C), lambda i: (0, 0)),
            pl.BlockSpec((C, C), lambda i: (0, 0)),
            pl.BlockSpec((C, H), lambda i: (0, 0)),
            pl.BlockSpec((1, H), lambda i: (0, 0)),
            pl.BlockSpec((1, H), lambda i: (0, 0)),
            pl.BlockSpec((1, H), lambda i: (0, 0)),
            pl.BlockSpec((H, C), lambda i: (0, 0)),
            pl.BlockSpec((1, C), lambda i: (0, 0)),
            pl.BlockSpec((1, 1, _BN), lambda i: (i, 0, 0)),
        ],
        out_specs=[
            pl.BlockSpec((_BN, C), lambda i: (i, 0)),
            pl.BlockSpec((3, C), lambda i: (0, 0)),
        ],
        out_shape=[
            jax.ShapeDtypeStruct((N, C), jnp.float32),
            jax.ShapeDtypeStruct((3, C), jnp.float32),
        ],
        scratch_shapes=[
            pltpu.VMEM((1, H), jnp.float32),
            pltpu.VMEM((1, H), jnp.float32),
        ],
    )(x1, m, s, w1, b1, bnw, bnb, w2, b2, nb3)


def _ln_body(x2_ref, gst_ref, nb_ref, lnw_ref, lnb_ref, out_ref):
    s1 = gst_ref[0:1, :]
    s2 = gst_ref[1:2, :]
    dg = gst_ref[2:3, :]
    norm = jnp.maximum(dg, 1.0) * C
    mean = s1 / norm
    var = (s2 - 2.0 * mean * s1 + mean * mean * dg * C) / norm
    inv = lax.rsqrt(var + EPS)

    nb = nb_ref[0, 0, :]
    oh = (nb[:, None] == lax.broadcasted_iota(jnp.int32, (1, C), 1)
          ).astype(jnp.float32)
    mean_n = jnp.sum(oh * mean, axis=1, keepdims=True)
    inv_n = jnp.sum(oh * inv, axis=1, keepdims=True)
    x2 = x2_ref[...]
    out_ref[...] = (x2 - mean_n) * inv_n * lnw_ref[...] + lnb_ref[...]


def _ln_apply(x2, gst, nb3, lnw, lnb):
    return pl.pallas_call(
        _ln_body,
        grid=(_NB,),
        in_specs=[
            pl.BlockSpec((_BN, C), lambda i: (i, 0)),
            pl.BlockSpec((3, C), lambda i: (0, 0)),
            pl.BlockSpec((1, 1, _BN), lambda i: (i, 0, 0)),
            pl.BlockSpec((1, C), lambda i: (0, 0)),
            pl.BlockSpec((1, C), lambda i: (0, 0)),
        ],
        out_specs=pl.BlockSpec((_BN, C), lambda i: (i, 0)),
        out_shape=jax.ShapeDtypeStruct((N, C), jnp.float32),
    )(x2, gst, nb3, lnw, lnb)


def _sc_edge_body(tg_hbm, ts_hbm, te_hbm, dst_hbm, src_hbm,
                  p0_hbm, p1_hbm,
                  dsti_v, srci_v, dg0, dg1, sg0, sg1, ds0, ds1,
                  g0, g1, s0, s1, t0, t1, m0, m1,
                  aggr_sh, semL, semS):
    dg_v = (dg0, dg1)
    sg_v = (sg0, sg1)
    ds_v = (ds0, ds1)
    g_v = (g0, g1)
    s_v = (s0, s1)
    t_v = (t0, t1)
    m_v = (m0, m1)
    cid = lax.axis_index("c")
    sid = lax.axis_index("s")
    wid = cid * _NS + sid
    cbase = wid * _CPW

    def _fill(buf, big, j):
        buf[...] = big[j >> 3, pl.ds((j & 7) * 16, 16)]

    def _zrow(r, _):
        for jj in range(C // 16):
            m_v[0][r, pl.ds(jj * 16, 16)] = jnp.zeros((16,), jnp.float32)
        return _

    lax.fori_loop(0, _CH, _zrow, None)
    zbase = sid * _ZONE
    for k in range(_ZONE // _CH):
        pltpu.sync_copy(m_v[0], aggr_sh.at[pl.ds(zbase + k * _CH, _CH)])

    @pl.when(sid == _NS - 1)
    def _():
        pltpu.sync_copy(m_v[0], aggr_sh.at[pl.ds(_NS * _ZONE, 16)])

    pltpu.sync_copy(dst_hbm.at[wid], dsti_v)
    pltpu.sync_copy(src_hbm.at[wid], srci_v)
    plsc.subcore_barrier()

    def _start_loads(j, b):
        pltpu.async_copy(tg_hbm.at[dg_v[b]], g_v[b], semL)
        pltpu.async_copy(ts_hbm.at[sg_v[b]], s_v[b], semL)
        pltpu.async_copy(te_hbm.at[pl.ds((cbase + j) * _CH, _CH)], t_v[b], semL)

    def _drain_loads(b):
        pltpu.make_async_copy(tg_hbm.at[pl.ds(0, _CH)], g_v[b], semL).wait()
        pltpu.make_async_copy(ts_hbm.at[pl.ds(0, _CH)], s_v[b], semL).wait()
        pltpu.make_async_copy(te_hbm.at[pl.ds(0, _CH)], t_v[b], semL).wait()

    def _compute(b):
        @plsc.parallel_loop(0, _CH, 1, unroll=2)
        def _edge(e):
            for jj in range(C // 16):
                o = jj * 16
                g = (g_v[b][e, pl.ds(o, 16)] + s_v[b][e, pl.ds(o, 16)]
                     + t_v[b][e, pl.ds(o, 16)])
                s = (g_v[b][e, pl.ds(C + o, 16)] + s_v[b][e, pl.ds(C + o, 16)]
                     + t_v[b][e, pl.ds(C + o, 16)])
                sig = 1.0 / (1.0 + jnp.exp(-g))
                t = jnp.exp(-jnp.abs(s))
                p = t * _LP[5] + _LP[4]
                for cf in (_LP[3], _LP[2], _LP[1], _LP[0]):
                    p = p * t + cf
                sp = jnp.maximum(s, 0.0) + p
                m_v[b][e, pl.ds(o, 16)] = sig * sp

    def _chunk(j, b):
        _drain_loads(b)

        @pl.when(j >= 2)
        def _():
            pltpu.make_async_copy(p0_hbm.at[pl.ds(0, _CH)], m_v[b], semS).wait()

        _fill(ds_v[b], dsti_v, j)
        _compute(b)

        @pl.when(j + 2 < _CPW)
        def _():
            _fill(dg_v[b], dsti_v, j + 2)
            _fill(sg_v[b], srci_v, j + 2)
            _start_loads(j + 2, b)

        pltpu.async_copy(m_v[b], aggr_sh.at[ds_v[b]], semS, add=True)

    _fill(dg0, dsti_v, 0)
    _fill(sg0, srci_v, 0)
    _start_loads(0, 0)
    _fill(dg1, dsti_v, 1)
    _fill(sg1, srci_v, 1)
    _start_loads(1, 1)

    def _pair(g, _):
        _chunk(2 * g, 0)
        _chunk(2 * g + 1, 1)
        return _

    lax.fori_loop(0, _CPW // 2, _pair, None)
    if _CPW % 2:
        _chunk(_CPW - 1, 0)
    pltpu.make_async_copy(p0_hbm.at[pl.ds(0, _CH)], m_v[0], semS).wait()
    pltpu.make_async_copy(p0_hbm.at[pl.ds(0, _CH)], m_v[1], semS).wait()
    plsc.subcore_barrier()

    @pl.when(cid == 0)
    def _():
        pltpu.sync_copy(aggr_sh.at[pl.ds(zbase, _ZONE)],
                        p0_hbm.at[pl.ds(zbase, _ZONE)])

        @pl.when(sid == _NS - 1)
        def _():
            pltpu.sync_copy(aggr_sh.at[pl.ds(_NS * _ZONE, 16)],
                            p0_hbm.at[pl.ds(_NS * _ZONE, 16)])

    @pl.when(cid == 1)
    def _():
        pltpu.sync_copy(aggr_sh.at[pl.ds(zbase, _ZONE)],
                        p1_hbm.at[pl.ds(zbase, _ZONE)])

        @pl.when(sid == _NS - 1)
        def _():
            pltpu.sync_copy(aggr_sh.at[pl.ds(_NS * _ZONE, 16)],
                            p1_hbm.at[pl.ds(_NS * _ZONE, 16)])


_sc_mesh = plsc.VectorSubcoreMesh(core_axis_name="c", subcore_axis_name="s",
                                  num_cores=_NC, num_subcores=_NS)

_sc_edge = functools.partial(
    pl.kernel, _sc_edge_body,
    out_type=[
        jax.ShapeDtypeStruct((N, C), jnp.float32),
        jax.ShapeDtypeStruct((N, C), jnp.float32),
    ],
    mesh=_sc_mesh,
    scratch_types=(
        [pltpu.VMEM((_IDXROWS, 128), jnp.int32)] * 2
        + [pltpu.VMEM((_CH,), jnp.int32)] * 6
        + [pltpu.VMEM((_CH, 2 * C), jnp.float32)] * 6
        + [pltpu.VMEM((_CH, C), jnp.float32)] * 2
        + [pltpu.VMEM_SHARED((N, C), jnp.float32),
           pltpu.SemaphoreType.DMA, pltpu.SemaphoreType.DMA]
    ),
)


def kernel(x, node_batch, edge_index, edge_attr, Wf, bf, Ws, bs, W1, b1,
           bn_w, bn_b, W2, b2, ln_w, ln_b):
    L = Wf.shape[0]
    pad = _IDXROWS * 128 - _EPW
    dst3 = jnp.pad(edge_index[1].astype(jnp.int32).reshape(_NW, _EPW),
                   ((0, 0), (0, pad))).reshape(_NW, _IDXROWS, 128)
    src3 = jnp.pad(edge_index[0].astype(jnp.int32).reshape(_NW, _EPW),
                   ((0, 0), (0, pad))).reshape(_NW, _IDXROWS, 128)
    nb3 = node_batch.astype(jnp.int32).reshape(_NB, 1, _BN)

    we_all = jnp.concatenate(
        [jnp.concatenate([Wf[l][2 * C:], Ws[l][2 * C:]], axis=1)
         for l in range(L)], axis=1)
    be_all = jnp.concatenate(
        [jnp.concatenate([bf[l], bs[l]]) for l in range(L)]).reshape(1, 3 * 2 * C)

    te = _edge_proj(edge_attr, we_all, be_all)

    sc_edge = _sc_edge()

    for l in range(L):
        wd = jnp.concatenate([Wf[l][:C], Ws[l][:C]], axis=1)
        wsr = jnp.concatenate([Wf[l][C:2 * C], Ws[l][C:2 * C]], axis=1)
        tg, ts = _node_proj(x, wd, wsr)
        p0, p1 = sc_edge(tg, ts, te[l], dst3, src3)
        x1, m, s = _stats(x, p0, p1)
        x2, gst = _mlp(x1, m, s, W1[l], b1[l].reshape(1, H),
                       bn_w[l].reshape(1, H), bn_b[l].reshape(1, H),
                       W2[l], b2[l].reshape(1, C), nb3)
        x = _ln_apply(x2, gst, nb3, ln_w[l].reshape(1, C),
                      ln_b[l].reshape(1, C))
    return x
